# Initial kernel scaffold; baseline (speedup 1.0000x reference)
import functools

import jax
import jax.numpy as jnp
from jax import lax
from jax.experimental import pallas as pl
from jax.experimental.pallas import tpu as pltpu
from jax.experimental.pallas import tpu_sc as plsc

NC = 2
NS = 16
L = 16
NW = NC * NS

EDGE_CHUNK = 80


def _proj_body(x_ref, w_ref, b_ref, h_ref, q_ref, k_ref, v_ref):
    d = x_ref.shape[1]
    p = jnp.dot(x_ref[...], w_ref[...], preferred_element_type=jnp.float32)
    p = p + b_ref[...]
    h_ref[...] = p[:, 0 * d:1 * d]
    q_ref[...] = p[:, 1 * d:2 * d]
    k_ref[...] = p[:, 2 * d:3 * d]
    v_ref[...] = p[:, 3 * d:4 * d]


def _add_body(h_ref, p0_ref, p1_ref, o_ref):
    o_ref[...] = h_ref[...] + p0_ref[0] + p1_ref[0]


def _make_edge_kernel(n_nodes, n_edges, d):
    epw = n_edges // NW
    nchunk = epw // EDGE_CHUNK
    rpt = n_nodes // NS
    c = EDGE_CHUNK

    mesh = plsc.VectorSubcoreMesh(core_axis_name="c", subcore_axis_name="s")

    @functools.partial(
        pl.kernel,
        out_type=jax.ShapeDtypeStruct((NC, n_nodes, d), jnp.float32),
        mesh=mesh,
        scratch_types=[
            pltpu.VMEM((c,), jnp.int32),
            pltpu.VMEM((c,), jnp.int32),
            pltpu.VMEM((c, d), jnp.float32),
            pltpu.VMEM((c, d), jnp.float32),
            pltpu.VMEM((c, d), jnp.float32),
            pltpu.VMEM_SHARED((n_nodes, d), jnp.float32),
            pltpu.SemaphoreType.DMA,
            pltpu.SemaphoreType.DMA,
            pltpu.SemaphoreType.DMA,
        ],
    )
    def edge_kernel(q_hbm, k_hbm, v_hbm, snd_hbm, rcv_hbm, zero_hbm, out_hbm,
                    idx_s, idx_r, qbuf, kbuf, vbuf, acc, sem_q, sem_k, sem_v):
        cid = lax.axis_index("c")
        sid = lax.axis_index("s")
        wid = sid * NC + cid

        pltpu.sync_copy(zero_hbm.at[pl.ds(sid * rpt, rpt)],
                        acc.at[pl.ds(sid * rpt, rpt)])
        plsc.subcore_barrier()

        def chunk_body(g, carry):
            base = wid * epw + g * c
            pltpu.sync_copy(snd_hbm.at[pl.ds(base, c)], idx_s)
            pltpu.sync_copy(rcv_hbm.at[pl.ds(base, c)], idx_r)
            cq = pltpu.async_copy(q_hbm.at[idx_r], qbuf, sem_q)
            ck = pltpu.async_copy(k_hbm.at[idx_s], kbuf, sem_k)
            cv = pltpu.async_copy(v_hbm.at[idx_s], vbuf, sem_v)
            cq.wait()
            ck.wait()
            cv.wait()

            def edge_body(i, carry2):
                for j in range(d // L):
                    sl = pl.ds(j * L, L)
                    x = qbuf[i, sl] + kbuf[i, sl]
                    eta = 1.0 / (1.0 + jnp.exp(-x))
                    vbuf[i, sl] = eta * vbuf[i, sl]
                return carry2

            lax.fori_loop(0, c, edge_body, 0)
            pltpu.sync_copy(vbuf, acc.at[idx_r], add=True)
            return carry

        lax.fori_loop(0, nchunk, chunk_body, 0)
        plsc.subcore_barrier()
        pltpu.sync_copy(acc.at[pl.ds(sid * rpt, rpt)],
                        out_hbm.at[cid, pl.ds(sid * rpt, rpt)])

    return edge_kernel


def kernel(node_features, senders, receivers, W_kernel, W_bias):
    n, d = node_features.shape
    e = senders.shape[0]
    senders = senders.astype(jnp.int32)
    receivers = receivers.astype(jnp.int32)

    blk = 500
    grid = n // blk
    h, q, k, v = pl.pallas_call(
        _proj_body,
        grid=(grid,),
        in_specs=[
            pl.BlockSpec((blk, d), lambda i: (i, 0)),
            pl.BlockSpec((d, 4 * d), lambda i: (0, 0)),
            pl.BlockSpec((1, 4 * d), lambda i: (0, 0)),
        ],
        out_specs=[pl.BlockSpec((blk, d), lambda i: (i, 0)) for _ in range(4)],
        out_shape=[jax.ShapeDtypeStruct((n, d), jnp.float32) for _ in range(4)],
    )(node_features, W_kernel, W_bias.reshape(1, 4 * d))

    zeros = jnp.zeros((n, d), jnp.float32)
    part = _make_edge_kernel(n, e, d)(q, k, v, senders, receivers, zeros)

    out = pl.pallas_call(
        _add_body,
        grid=(grid,),
        in_specs=[
            pl.BlockSpec((blk, d), lambda i: (i, 0)),
            pl.BlockSpec((1, blk, d), lambda i: (0, i, 0)),
            pl.BlockSpec((1, blk, d), lambda i: (1, i, 0)),
        ],
        out_specs=pl.BlockSpec((blk, d), lambda i: (i, 0)),
        out_shape=jax.ShapeDtypeStruct((n, d), jnp.float32),
    )(h, part, part)
    return out


# speedup vs baseline: 5.8317x; 5.8317x over previous
import functools

import jax
import jax.numpy as jnp
from jax import lax
from jax.experimental import pallas as pl
from jax.experimental.pallas import tpu as pltpu
from jax.experimental.pallas import tpu_sc as plsc

NC = 2
NS = 16
L = 16
NW = NC * NS

EDGE_CHUNK = 80


def _proj_body(x_ref, w_ref, b_ref, h_ref, q_ref, k_ref, v_ref):
    d = x_ref.shape[1]
    p = jnp.dot(x_ref[...], w_ref[...], preferred_element_type=jnp.float32)
    p = p + b_ref[...]
    h_ref[...] = p[:, 0 * d:1 * d]
    q_ref[...] = p[:, 1 * d:2 * d]
    k_ref[...] = p[:, 2 * d:3 * d]
    v_ref[...] = p[:, 3 * d:4 * d]


def _add_body(h_ref, p0_ref, p1_ref, o_ref):
    o_ref[...] = h_ref[...] + p0_ref[0] + p1_ref[0]


def _make_edge_kernel(n_nodes, n_edges, d):
    epw = n_edges // NW
    nchunk = epw // EDGE_CHUNK
    rpt = (n_nodes // NS) // 8 * 8
    tail = n_nodes - rpt * NS
    c = EDGE_CHUNK

    mesh = plsc.VectorSubcoreMesh(core_axis_name="c", subcore_axis_name="s")

    @functools.partial(
        pl.kernel,
        out_type=jax.ShapeDtypeStruct((NC, n_nodes, d), jnp.float32),
        mesh=mesh,
        scratch_types=[
            pltpu.VMEM((c,), jnp.int32),
            pltpu.VMEM((c,), jnp.int32),
            pltpu.VMEM((c, d), jnp.float32),
            pltpu.VMEM((c, d), jnp.float32),
            pltpu.VMEM((c, d), jnp.float32),
            pltpu.VMEM_SHARED((n_nodes, d), jnp.float32),
            pltpu.SemaphoreType.DMA,
            pltpu.SemaphoreType.DMA,
            pltpu.SemaphoreType.DMA,
        ],
    )
    def edge_kernel(q_hbm, k_hbm, v_hbm, snd_hbm, rcv_hbm, zero_hbm, out_hbm,
                    idx_s, idx_r, qbuf, kbuf, vbuf, acc, sem_q, sem_k, sem_v):
        cid = lax.axis_index("c")
        sid = lax.axis_index("s")
        wid = sid * NC + cid

        pltpu.sync_copy(zero_hbm.at[pl.ds(sid * rpt, rpt)],
                        acc.at[pl.ds(sid * rpt, rpt)])
        if tail:
            @pl.when(sid == NS - 1)
            def _():
                pltpu.sync_copy(zero_hbm.at[pl.ds(rpt * NS, tail)],
                                acc.at[pl.ds(rpt * NS, tail)])
        plsc.subcore_barrier()

        def chunk_body(g, carry):
            base = wid * epw + g * c
            pltpu.sync_copy(snd_hbm.at[pl.ds(base, c)], idx_s)
            pltpu.sync_copy(rcv_hbm.at[pl.ds(base, c)], idx_r)
            cq = pltpu.async_copy(q_hbm.at[idx_r], qbuf, sem_q)
            ck = pltpu.async_copy(k_hbm.at[idx_s], kbuf, sem_k)
            cv = pltpu.async_copy(v_hbm.at[idx_s], vbuf, sem_v)
            cq.wait()
            ck.wait()
            cv.wait()

            def edge_body(i, carry2):
                for j in range(d // L):
                    sl = pl.ds(j * L, L)
                    x = qbuf[i, sl] + kbuf[i, sl]
                    eta = 1.0 / (1.0 + jnp.exp(-x))
                    vbuf[i, sl] = eta * vbuf[i, sl]
                return carry2

            lax.fori_loop(0, c, edge_body, 0)
            pltpu.sync_copy(vbuf, acc.at[idx_r], add=True)
            return carry

        lax.fori_loop(0, nchunk, chunk_body, 0)
        plsc.subcore_barrier()
        pltpu.sync_copy(acc.at[pl.ds(sid * rpt, rpt)],
                        out_hbm.at[cid, pl.ds(sid * rpt, rpt)])
        if tail:
            @pl.when(sid == NS - 1)
            def _():
                pltpu.sync_copy(acc.at[pl.ds(rpt * NS, tail)],
                                out_hbm.at[cid, pl.ds(rpt * NS, tail)])

    return edge_kernel


def kernel(node_features, senders, receivers, W_kernel, W_bias):
    n, d = node_features.shape
    e = senders.shape[0]
    senders = senders.astype(jnp.int32)
    receivers = receivers.astype(jnp.int32)

    blk = 1000
    grid = n // blk
    h, q, k, v = pl.pallas_call(
        _proj_body,
        grid=(grid,),
        in_specs=[
            pl.BlockSpec((blk, d), lambda i: (i, 0)),
            pl.BlockSpec((d, 4 * d), lambda i: (0, 0)),
            pl.BlockSpec((1, 4 * d), lambda i: (0, 0)),
        ],
        out_specs=[pl.BlockSpec((blk, d), lambda i: (i, 0)) for _ in range(4)],
        out_shape=[jax.ShapeDtypeStruct((n, d), jnp.float32) for _ in range(4)],
    )(node_features, W_kernel, W_bias.reshape(1, 4 * d))

    zeros = jnp.zeros((n, d), jnp.float32)
    part = _make_edge_kernel(n, e, d)(q, k, v, senders, receivers, zeros)

    out = pl.pallas_call(
        _add_body,
        grid=(grid,),
        in_specs=[
            pl.BlockSpec((blk, d), lambda i: (i, 0)),
            pl.BlockSpec((1, blk, d), lambda i: (0, i, 0)),
            pl.BlockSpec((1, blk, d), lambda i: (1, i, 0)),
        ],
        out_specs=pl.BlockSpec((blk, d), lambda i: (i, 0)),
        out_shape=jax.ShapeDtypeStruct((n, d), jnp.float32),
    )(h, part, part)
    return out


# speedup vs baseline: 9.4386x; 1.6185x over previous
import functools

import jax
import jax.numpy as jnp
from jax import lax
from jax.experimental import pallas as pl
from jax.experimental.pallas import tpu as pltpu
from jax.experimental.pallas import tpu_sc as plsc

NC = 2
NS = 16
L = 16
NW = NC * NS

EDGE_CHUNK = 40


def _proj_body(x_ref, w_ref, b_ref, h_ref, q_ref, k_ref, v_ref):
    d = x_ref.shape[1]
    p = jnp.dot(x_ref[...], w_ref[...], preferred_element_type=jnp.float32)
    p = p + b_ref[...]
    h_ref[...] = p[:, 0 * d:1 * d]
    q_ref[...] = p[:, 1 * d:2 * d]
    k_ref[...] = p[:, 2 * d:3 * d]
    v_ref[...] = p[:, 3 * d:4 * d]


def _add_body(h_ref, p0_ref, p1_ref, o_ref):
    o_ref[...] = h_ref[...] + p0_ref[0] + p1_ref[0]


def _make_edge_kernel(n_nodes, n_edges, d):
    epw = n_edges // NW
    c = EDGE_CHUNK
    nchunk = epw // c
    assert nchunk % 4 == 2 and nchunk >= 6
    nquads = (nchunk - 2) // 4
    rpt = (n_nodes // NS) // 8 * 8
    tail = n_nodes - rpt * NS

    mesh = plsc.VectorSubcoreMesh(core_axis_name="c", subcore_axis_name="s")

    @functools.partial(
        pl.kernel,
        out_type=jax.ShapeDtypeStruct((NC, n_nodes, d), jnp.float32),
        mesh=mesh,
        scratch_types=[
            pltpu.VMEM((2, c), jnp.int32),
            pltpu.VMEM((2, c), jnp.int32),
            pltpu.VMEM((2, c), jnp.int32),
            pltpu.VMEM((2, c), jnp.int32),
            pltpu.VMEM((c, d), jnp.float32),
            pltpu.VMEM((c, d), jnp.float32),
            pltpu.VMEM((c, d), jnp.float32),
            pltpu.VMEM((c, d), jnp.float32),
            pltpu.VMEM((c, d), jnp.float32),
            pltpu.VMEM((c, d), jnp.float32),
            pltpu.VMEM_SHARED((n_nodes, d), jnp.float32),
            pltpu.SemaphoreType.DMA,
            pltpu.SemaphoreType.DMA,
            pltpu.SemaphoreType.DMA,
            pltpu.SemaphoreType.DMA,
            pltpu.SemaphoreType.DMA,
            pltpu.SemaphoreType.DMA,
        ],
    )
    def edge_kernel(q_hbm, k_hbm, v_hbm, sr_hbm, zero_hbm, out_hbm,
                    s0, s1, s2, s3, qa, ka, va, qb, kb, vb, acc,
                    ss0, ss1, ss2, ss3, sem_a, sem_b):
        cid = lax.axis_index("c")
        sid = lax.axis_index("s")
        wid = sid * NC + cid
        srs = ((s0, ss0), (s1, ss1), (s2, ss2), (s3, ss3))
        sets = ((qa, ka, va, sem_a), (qb, kb, vb, sem_b))

        pltpu.sync_copy(zero_hbm.at[pl.ds(sid * rpt, rpt)],
                        acc.at[pl.ds(sid * rpt, rpt)])
        if tail:
            @pl.when(sid == NS - 1)
            def _():
                pltpu.sync_copy(zero_hbm.at[pl.ds(rpt * NS, tail)],
                                acc.at[pl.ds(rpt * NS, tail)])
        plsc.subcore_barrier()

        def fire_sr(g, slot):
            sr, sem = srs[slot]
            pltpu.async_copy(sr_hbm.at[wid, g], sr, sem)

        def wait_sr(slot):
            sr, sem = srs[slot]
            pltpu.make_async_copy(sr_hbm.at[wid, 0], sr, sem).wait()

        def fire_gather(slot, st):
            sr, _ = srs[slot]
            qx, kx, vx, sem = sets[st]
            pltpu.async_copy(q_hbm.at[sr.at[1]], qx, sem)
            pltpu.async_copy(k_hbm.at[sr.at[0]], kx, sem)
            pltpu.async_copy(v_hbm.at[sr.at[0]], vx, sem)

        def process(slot, st):
            sr, _ = srs[slot]
            qx, kx, vx, sem = sets[st]
            dummy = q_hbm.at[pl.ds(0, c)]
            pltpu.make_async_copy(dummy, qx, sem).wait()
            pltpu.make_async_copy(dummy, kx, sem).wait()
            pltpu.make_async_copy(dummy, vx, sem).wait()

            def edge_body(i, carry2):
                for j in range(d // L):
                    sl = pl.ds(j * L, L)
                    x = qx[i, sl] + kx[i, sl]
                    eta = 1.0 / (1.0 + jnp.exp(-x))
                    vx[i, sl] = eta * vx[i, sl]
                return carry2

            lax.fori_loop(0, c, edge_body, 0)
            pltpu.sync_copy(vx, acc.at[sr.at[1]], add=True)

        fire_sr(0, 0)
        fire_sr(1, 1)
        fire_sr(2, 2)
        fire_sr(3, 3)
        wait_sr(0)
        fire_gather(0, 0)

        def quad_body(i, carry):
            c0 = 4 * i
            wait_sr(1)
            fire_gather(1, 1)
            process(0, 0)
            fire_sr(c0 + 4, 0)
            wait_sr(2)
            fire_gather(2, 0)
            process(1, 1)
            fire_sr(c0 + 5, 1)
            wait_sr(3)
            fire_gather(3, 1)
            process(2, 0)

            @pl.when(c0 + 6 < nchunk)
            def _():
                fire_sr(c0 + 6, 2)

            wait_sr(0)
            fire_gather(0, 0)
            process(3, 1)

            @pl.when(c0 + 7 < nchunk)
            def _():
                fire_sr(c0 + 7, 3)

            return carry

        lax.fori_loop(0, nquads, quad_body, 0)
        wait_sr(1)
        fire_gather(1, 1)
        process(0, 0)
        process(1, 1)

        plsc.subcore_barrier()
        pltpu.sync_copy(acc.at[pl.ds(sid * rpt, rpt)],
                        out_hbm.at[cid, pl.ds(sid * rpt, rpt)])
        if tail:
            @pl.when(sid == NS - 1)
            def _():
                pltpu.sync_copy(acc.at[pl.ds(rpt * NS, tail)],
                                out_hbm.at[cid, pl.ds(rpt * NS, tail)])

    return edge_kernel


def kernel(node_features, senders, receivers, W_kernel, W_bias):
    n, d = node_features.shape
    e = senders.shape[0]
    senders = senders.astype(jnp.int32)
    receivers = receivers.astype(jnp.int32)

    blk = 1000
    grid = n // blk
    h, q, k, v = pl.pallas_call(
        _proj_body,
        grid=(grid,),
        in_specs=[
            pl.BlockSpec((blk, d), lambda i: (i, 0)),
            pl.BlockSpec((d, 4 * d), lambda i: (0, 0)),
            pl.BlockSpec((1, 4 * d), lambda i: (0, 0)),
        ],
        out_specs=[pl.BlockSpec((blk, d), lambda i: (i, 0)) for _ in range(4)],
        out_shape=[jax.ShapeDtypeStruct((n, d), jnp.float32) for _ in range(4)],
    )(node_features, W_kernel, W_bias.reshape(1, 4 * d))

    zeros = jnp.zeros((n, d), jnp.float32)
    epw = e // NW
    nchunk = epw // EDGE_CHUNK
    sr = jnp.stack(
        (senders.reshape(NW, nchunk, EDGE_CHUNK),
         receivers.reshape(NW, nchunk, EDGE_CHUNK)), axis=2)
    part = _make_edge_kernel(n, e, d)(q, k, v, sr, zeros)

    out = pl.pallas_call(
        _add_body,
        grid=(grid,),
        in_specs=[
            pl.BlockSpec((blk, d), lambda i: (i, 0)),
            pl.BlockSpec((1, blk, d), lambda i: (0, i, 0)),
            pl.BlockSpec((1, blk, d), lambda i: (1, i, 0)),
        ],
        out_specs=pl.BlockSpec((blk, d), lambda i: (i, 0)),
        out_shape=jax.ShapeDtypeStruct((n, d), jnp.float32),
    )(h, part, part)
    return out
